# baseline (speedup 1.0000x reference)
import jax
import jax.numpy as jnp
from jax.experimental import pallas as pl
from jax.experimental.pallas import tpu as pltpu

B = 1024
BB = 256
F = 81920
FB = 8192


def _nnue_kernel(white_ref, black_ref, turn_ref, score_ref,
                 w0_ref, b0_ref, w1_ref, b1_ref, w2_ref, b2_ref,
                 loss_ref, accw_ref, accb_ref):
    i = pl.program_id(1)
    nsteps = pl.num_programs(1)

    @pl.when(i == 0)
    def _init():
        accw_ref[...] = jnp.zeros_like(accw_ref)
        accb_ref[...] = jnp.zeros_like(accb_ref)

    dn = (((1,), (1,)), ((), ()))
    w0 = w0_ref[...]
    wpart = jax.lax.dot_general(white_ref[...], w0, dn,
                                preferred_element_type=jnp.float32)
    bpart = jax.lax.dot_general(black_ref[...], w0, dn,
                                preferred_element_type=jnp.float32)
    accw_ref[...] += wpart
    accb_ref[...] += bpart

    @pl.when(i == nsteps - 1)
    def _epilogue():
        b0 = b0_ref[...]
        w = accw_ref[...] + b0
        b = accb_ref[...] + b0
        turn = turn_ref[...]
        wb = jnp.concatenate([w, b], axis=1)
        bw = jnp.concatenate([b, w], axis=1)
        accum = turn * wb + (1.0 - turn) * bw
        l1_x = jnp.clip(accum, 0.0, 1.0)
        dn2 = (((1,), (1,)), ((), ()))
        l2 = jax.lax.dot_general(l1_x, w1_ref[...], dn2,
                                 preferred_element_type=jnp.float32) + b1_ref[...]
        l2_x = jnp.clip(l2, 0.0, 1.0)
        model = jnp.sum(l2_x * w2_ref[...], axis=1,
                        keepdims=True) + b2_ref[...]
        wdl_model = jax.nn.sigmoid(model / 400.0)
        wdl_target = jax.nn.sigmoid(score_ref[...] / 400.0)
        loss_ref[...] = (wdl_model - wdl_target) ** 2


@jax.jit
def _nnue(white_features, black_features, turn, score,
          W0, b0, W1, b1, W2, b2):
    grid = (B // BB, F // FB)
    return pl.pallas_call(
        _nnue_kernel,
        grid=grid,
        in_specs=[
            pl.BlockSpec((BB, FB), lambda b, i: (b, i)),
            pl.BlockSpec((BB, FB), lambda b, i: (b, i)),
            pl.BlockSpec((BB, 1), lambda b, i: (b, 0)),
            pl.BlockSpec((BB, 1), lambda b, i: (b, 0)),
            pl.BlockSpec((4, FB), lambda b, i: (0, i)),
            pl.BlockSpec((1, 4), lambda b, i: (0, 0)),
            pl.BlockSpec((8, 8), lambda b, i: (0, 0)),
            pl.BlockSpec((1, 8), lambda b, i: (0, 0)),
            pl.BlockSpec((1, 8), lambda b, i: (0, 0)),
            pl.BlockSpec((1, 1), lambda b, i: (0, 0)),
        ],
        out_specs=pl.BlockSpec((BB, 1), lambda b, i: (b, 0)),
        out_shape=jax.ShapeDtypeStruct((B, 1), jnp.float32),
        scratch_shapes=[pltpu.VMEM((BB, 4), jnp.float32),
                        pltpu.VMEM((BB, 4), jnp.float32)],
    )(white_features, black_features, turn, score,
      W0, b0, W1, b1, W2, b2)


def kernel(white_features, black_features, turn, score, result,
           W0, b0, W1, b1, W2, b2):
    del result
    return _nnue(white_features, black_features, turn, score,
                 W0, b0.reshape(1, 4), W1, b1.reshape(1, 8),
                 W2.reshape(1, 8), b2.reshape(1, 1))


# speedup vs baseline: 1.0026x; 1.0026x over previous
import jax
import jax.numpy as jnp
from jax.experimental import pallas as pl
from jax.experimental.pallas import tpu as pltpu

B = 1024
F = 81920
FB = 2048


def _nnue_kernel(white_ref, black_ref, turn_ref, score_ref,
                 w0_ref, b0_ref, w1_ref, b1_ref, w2_ref, b2_ref,
                 loss_ref, accw_ref, accb_ref):
    i = pl.program_id(0)
    nsteps = pl.num_programs(0)

    @pl.when(i == 0)
    def _init():
        accw_ref[...] = jnp.zeros_like(accw_ref)
        accb_ref[...] = jnp.zeros_like(accb_ref)

    dn = (((1,), (1,)), ((), ()))
    w0 = w0_ref[...]
    wpart = jax.lax.dot_general(white_ref[...], w0, dn,
                                preferred_element_type=jnp.float32)
    bpart = jax.lax.dot_general(black_ref[...], w0, dn,
                                preferred_element_type=jnp.float32)
    accw_ref[...] += wpart
    accb_ref[...] += bpart

    @pl.when(i == nsteps - 1)
    def _epilogue():
        b0 = b0_ref[...]
        w = accw_ref[...] + b0
        b = accb_ref[...] + b0
        turn = turn_ref[...]
        wb = jnp.concatenate([w, b], axis=1)
        bw = jnp.concatenate([b, w], axis=1)
        accum = turn * wb + (1.0 - turn) * bw
        l1_x = jnp.clip(accum, 0.0, 1.0)
        dn2 = (((1,), (1,)), ((), ()))
        l2 = jax.lax.dot_general(l1_x, w1_ref[...], dn2,
                                 preferred_element_type=jnp.float32) + b1_ref[...]
        l2_x = jnp.clip(l2, 0.0, 1.0)
        model = jnp.sum(l2_x * w2_ref[...], axis=1,
                        keepdims=True) + b2_ref[...]
        wdl_model = jax.nn.sigmoid(model / 400.0)
        wdl_target = jax.nn.sigmoid(score_ref[...] / 400.0)
        loss_ref[...] = (wdl_model - wdl_target) ** 2


@jax.jit
def _nnue(white_features, black_features, turn, score,
          W0, b0, W1, b1, W2, b2):
    grid = (F // FB,)
    return pl.pallas_call(
        _nnue_kernel,
        grid=grid,
        in_specs=[
            pl.BlockSpec((B, FB), lambda i: (0, i)),
            pl.BlockSpec((B, FB), lambda i: (0, i)),
            pl.BlockSpec((B, 1), lambda i: (0, 0)),
            pl.BlockSpec((B, 1), lambda i: (0, 0)),
            pl.BlockSpec((4, FB), lambda i: (0, i)),
            pl.BlockSpec((1, 4), lambda i: (0, 0)),
            pl.BlockSpec((8, 8), lambda i: (0, 0)),
            pl.BlockSpec((1, 8), lambda i: (0, 0)),
            pl.BlockSpec((1, 8), lambda i: (0, 0)),
            pl.BlockSpec((1, 1), lambda i: (0, 0)),
        ],
        out_specs=pl.BlockSpec((B, 1), lambda i: (0, 0)),
        out_shape=jax.ShapeDtypeStruct((B, 1), jnp.float32),
        scratch_shapes=[pltpu.VMEM((B, 4), jnp.float32),
                        pltpu.VMEM((B, 4), jnp.float32)],
    )(white_features, black_features, turn, score,
      W0, b0, W1, b1, W2, b2)


def kernel(white_features, black_features, turn, score, result,
           W0, b0, W1, b1, W2, b2):
    del result
    return _nnue(white_features, black_features, turn, score,
                 W0, b0.reshape(1, 4), W1, b1.reshape(1, 8),
                 W2.reshape(1, 8), b2.reshape(1, 1))
